# baseline (speedup 1.0000x reference)
import functools

import jax
import jax.numpy as jnp
from jax import lax
from jax.experimental import pallas as pl
from jax.experimental.pallas import tpu as pltpu
from jax.experimental.pallas import tpu_sc as plsc

N = 10000
E = 320000
H = 128
G = 64
L = 5
BN_EPS = 1e-5

NC = 2
NS = 16
NW = NC * NS
CHUNK = 112
NCHUNK = 91
EPW = NCHUNK * CHUNK
E_PAD = NW * EPW
N_PAD = 10240
ROWS_PER_TILE = N_PAD // NS


def _segsum_body(h_hbm, src_hbm, dst_hbm, out_hbm,
                 src_v, dst_v, rows_a, rows_b, agg_sh, sem_a, sem_b):
    cid = lax.axis_index("c")
    sid = lax.axis_index("s")
    wid = cid * NS + sid

    pltpu.async_copy(src_hbm.at[wid], src_v, sem_a)
    pltpu.async_copy(dst_hbm.at[wid], dst_v, sem_b)

    def _zero_row(i, carry):
        for c in range(H // 16):
            rows_a[i, pl.ds(c * 16, 16)] = jnp.zeros((16,), jnp.float32)
        return carry
    lax.fori_loop(0, CHUNK, _zero_row, 0)
    row0 = sid * ROWS_PER_TILE
    for k in range(ROWS_PER_TILE // CHUNK):
        pltpu.sync_copy(rows_a, agg_sh.at[pl.ds(row0 + k * CHUNK, CHUNK)])
    rem = ROWS_PER_TILE - (ROWS_PER_TILE // CHUNK) * CHUNK
    if rem:
        pltpu.sync_copy(
            rows_a.at[pl.ds(0, rem)],
            agg_sh.at[pl.ds(row0 + (ROWS_PER_TILE // CHUNK) * CHUNK, rem)])
    pltpu.make_async_copy(src_hbm.at[wid], src_v, sem_a).wait()
    pltpu.make_async_copy(dst_hbm.at[wid], dst_v, sem_b).wait()
    plsc.subcore_barrier()

    pltpu.async_copy(h_hbm.at[src_v.at[0]], rows_a, sem_a)

    def _pair(jj, carry):
        j0 = 2 * jj
        pltpu.make_async_copy(h_hbm.at[src_v.at[j0]], rows_a, sem_a).wait()
        pltpu.async_copy(h_hbm.at[src_v.at[j0 + 1]], rows_b, sem_b)
        pltpu.sync_copy(rows_a, agg_sh.at[dst_v.at[j0]], add=True)
        pltpu.make_async_copy(h_hbm.at[src_v.at[j0 + 1]], rows_b, sem_b).wait()
        pltpu.async_copy(h_hbm.at[src_v.at[j0 + 2]], rows_a, sem_a)
        pltpu.sync_copy(rows_b, agg_sh.at[dst_v.at[j0 + 1]], add=True)
        return carry
    lax.fori_loop(0, (NCHUNK - 1) // 2, _pair, 0)
    pltpu.make_async_copy(h_hbm.at[src_v.at[NCHUNK - 1]], rows_a, sem_a).wait()
    pltpu.sync_copy(rows_a, agg_sh.at[dst_v.at[NCHUNK - 1]], add=True)
    plsc.subcore_barrier()

    pltpu.sync_copy(agg_sh.at[pl.ds(row0, ROWS_PER_TILE)],
                    out_hbm.at[cid, pl.ds(row0, ROWS_PER_TILE)])


@functools.cache
def _segsum_call():
    return pl.kernel(
        _segsum_body,
        compiler_params=pltpu.CompilerParams(use_tc_tiling_on_sc=False),
        out_type=jax.ShapeDtypeStruct((NC, N_PAD, H), jnp.float32),
        mesh=plsc.VectorSubcoreMesh(core_axis_name="c", subcore_axis_name="s",
                                    num_cores=NC, num_subcores=NS),
        scratch_types=[
            pltpu.VMEM((NCHUNK, CHUNK), jnp.int32),
            pltpu.VMEM((NCHUNK, CHUNK), jnp.int32),
            pltpu.VMEM((CHUNK, H), jnp.float32),
            pltpu.VMEM((CHUNK, H), jnp.float32),
            pltpu.VMEM_SHARED((N_PAD, H), jnp.float32),
            pltpu.SemaphoreType.DMA,
            pltpu.SemaphoreType.DMA,
        ],
    )


def _pool(batch2, h):
    onehot = (batch2 == lax.broadcasted_iota(jnp.int32, (N, G), 1))
    return lax.dot_general(onehot.astype(jnp.float32), h,
                           (((0,), (0,)), ((), ())),
                           preferred_element_type=jnp.float32)


def _inproj_body(x_ref, w_ref, b_ref, batch_ref, h_ref, pool_ref):
    h = jnp.dot(x_ref[...], w_ref[...],
                preferred_element_type=jnp.float32) + b_ref[...]
    h_ref[...] = h
    pool_ref[...] = _pool(batch_ref[...], h)


def _bn_relu(t, g, b):
    m = jnp.mean(t, axis=0, keepdims=True)
    v = jnp.mean(t * t, axis=0, keepdims=True) - m * m
    return jnp.maximum((t - m) * lax.rsqrt(v + BN_EPS) * g + b, 0.0)


def _layer_body(h_ref, agg_ref, eps_ref, w1_ref, b1_ref, g1_ref, be1_ref,
                w2_ref, b2_ref, gbn_ref, bbn_ref, batch_ref,
                hout_ref, pool_ref):
    z = (1.0 + eps_ref[0, 0]) * h_ref[...] + agg_ref[0, :N, :] + agg_ref[1, :N, :]
    t = jnp.dot(z, w1_ref[...], preferred_element_type=jnp.float32) + b1_ref[...]
    t = _bn_relu(t, g1_ref[...], be1_ref[...])
    u = jnp.dot(t, w2_ref[...], preferred_element_type=jnp.float32) + b2_ref[...]
    hn = _bn_relu(u, gbn_ref[...], bbn_ref[...])
    hout_ref[...] = hn
    pool_ref[...] = _pool(batch_ref[...], hn)


def _final_body(pools_ref, w_ref, b_ref, out_ref):
    p = pools_ref[...]
    pc = jnp.concatenate([p[i] for i in range(L + 1)], axis=1)
    out_ref[...] = jnp.dot(pc, w_ref[...],
                           preferred_element_type=jnp.float32) + b_ref[...]


@functools.cache
def _inproj_call():
    return pl.pallas_call(
        _inproj_body,
        out_shape=(jax.ShapeDtypeStruct((N, H), jnp.float32),
                   jax.ShapeDtypeStruct((G, H), jnp.float32)),
    )


@functools.cache
def _layer_call():
    return pl.pallas_call(
        _layer_body,
        out_shape=(jax.ShapeDtypeStruct((N, H), jnp.float32),
                   jax.ShapeDtypeStruct((G, H), jnp.float32)),
    )


@functools.cache
def _final_call():
    return pl.pallas_call(
        _final_body,
        out_shape=jax.ShapeDtypeStruct((G, H), jnp.float32),
    )


def kernel(x, edge_index, batch, params):
    pad = E_PAD - E
    src = jnp.concatenate(
        [edge_index[0], jnp.zeros((pad,), jnp.int32)]).reshape(NW, NCHUNK, CHUNK)
    dst = jnp.concatenate(
        [edge_index[1], jnp.full((pad,), N, jnp.int32)]).reshape(NW, NCHUNK, CHUNK)
    batch2 = batch.reshape(N, 1)

    h, pool0 = _inproj_call()(
        x, params['in_proj']['w'], params['in_proj']['b'].reshape(1, H), batch2)
    pools = [pool0]
    for lp in params['layers']:
        agg2 = _segsum_call()(h, src, dst)
        h, p = _layer_call()(
            h, agg2, lp['eps'].reshape(1, 1),
            lp['w1'], lp['b1'].reshape(1, 2 * H),
            lp['g1'].reshape(1, 2 * H), lp['be1'].reshape(1, 2 * H),
            lp['w2'], lp['b2'].reshape(1, H),
            lp['g_bn'].reshape(1, H), lp['b_bn'].reshape(1, H), batch2)
        pools.append(p)

    return _final_call()(jnp.stack(pools),
                         params['out_proj']['w'],
                         params['out_proj']['b'].reshape(1, H))


# speedup vs baseline: 2.5369x; 2.5369x over previous
import functools

import jax
import jax.numpy as jnp
from jax import lax
from jax.experimental import pallas as pl
from jax.experimental.pallas import tpu as pltpu
from jax.experimental.pallas import tpu_sc as plsc

N = 10000
E = 320000
H = 128
G = 64
L = 5
BN_EPS = 1e-5

NC = 2
NS = 16
NW = NC * NS
CHUNK = 80
NCHUNK = 125
EPW = NCHUNK * CHUNK
E_PAD = NW * EPW
N_PAD = 10240
ROWS_PER_TILE = N_PAD // NS


def _segsum_body(h_hbm, src_hbm, dst_hbm, out_hbm,
                 src_v, dst_v, rows_a, rows_b, agg_sh, sem_a, sem_b):
    cid = lax.axis_index("c")
    sid = lax.axis_index("s")
    wid = cid * NS + sid

    pltpu.async_copy(src_hbm.at[wid], src_v, sem_a)
    pltpu.async_copy(dst_hbm.at[wid], dst_v, sem_b)

    def _zero_row(i, carry):
        for c in range(H // 16):
            rows_a[i, pl.ds(c * 16, 16)] = jnp.zeros((16,), jnp.float32)
        return carry
    lax.fori_loop(0, CHUNK, _zero_row, 0)
    row0 = sid * ROWS_PER_TILE
    for k in range(ROWS_PER_TILE // CHUNK):
        pltpu.sync_copy(rows_a, agg_sh.at[pl.ds(row0 + k * CHUNK, CHUNK)])
    rem = ROWS_PER_TILE - (ROWS_PER_TILE // CHUNK) * CHUNK
    if rem:
        pltpu.sync_copy(
            rows_a.at[pl.ds(0, rem)],
            agg_sh.at[pl.ds(row0 + (ROWS_PER_TILE // CHUNK) * CHUNK, rem)])
    pltpu.make_async_copy(src_hbm.at[wid], src_v, sem_a).wait()
    pltpu.make_async_copy(dst_hbm.at[wid], dst_v, sem_b).wait()
    plsc.subcore_barrier()

    pltpu.async_copy(h_hbm.at[src_v.at[0]], rows_a, sem_a)

    def _pair(jj, carry):
        j0 = 2 * jj
        pltpu.make_async_copy(h_hbm.at[src_v.at[j0]], rows_a, sem_a).wait()
        pltpu.async_copy(h_hbm.at[src_v.at[j0 + 1]], rows_b, sem_b)
        pltpu.sync_copy(rows_a, agg_sh.at[dst_v.at[j0]], add=True)
        pltpu.make_async_copy(h_hbm.at[src_v.at[j0 + 1]], rows_b, sem_b).wait()
        pltpu.async_copy(h_hbm.at[src_v.at[j0 + 2]], rows_a, sem_a)
        pltpu.sync_copy(rows_b, agg_sh.at[dst_v.at[j0 + 1]], add=True)
        return carry
    lax.fori_loop(0, (NCHUNK - 1) // 2, _pair, 0)
    pltpu.make_async_copy(h_hbm.at[src_v.at[NCHUNK - 1]], rows_a, sem_a).wait()
    pltpu.sync_copy(rows_a, agg_sh.at[dst_v.at[NCHUNK - 1]], add=True)
    plsc.subcore_barrier()

    pltpu.sync_copy(agg_sh.at[pl.ds(row0, ROWS_PER_TILE)],
                    out_hbm.at[cid, pl.ds(row0, ROWS_PER_TILE)])


@functools.cache
def _segsum_call():
    return pl.kernel(
        _segsum_body,
        compiler_params=pltpu.CompilerParams(use_tc_tiling_on_sc=False),
        out_type=jax.ShapeDtypeStruct((NC, N_PAD, H), jnp.float32),
        mesh=plsc.VectorSubcoreMesh(core_axis_name="c", subcore_axis_name="s",
                                    num_cores=NC, num_subcores=NS),
        scratch_types=[
            pltpu.VMEM((NCHUNK, CHUNK), jnp.int32),
            pltpu.VMEM((NCHUNK, CHUNK), jnp.int32),
            pltpu.VMEM((CHUNK, H), jnp.float32),
            pltpu.VMEM((CHUNK, H), jnp.float32),
            pltpu.VMEM_SHARED((N_PAD, H), jnp.float32),
            pltpu.SemaphoreType.DMA,
            pltpu.SemaphoreType.DMA,
        ],
    )


def _pool(batch2, h):
    onehot = (batch2 == lax.broadcasted_iota(jnp.int32, (N, G), 1))
    return lax.dot_general(onehot.astype(jnp.float32), h,
                           (((0,), (0,)), ((), ())),
                           preferred_element_type=jnp.float32)


def _inproj_body(x_ref, w_ref, b_ref, batch_ref, h_ref, pool_ref):
    h = jnp.dot(x_ref[...], w_ref[...],
                preferred_element_type=jnp.float32) + b_ref[...]
    h_ref[...] = h
    pool_ref[...] = _pool(batch_ref[...], h)


def _bn_relu(t, g, b):
    m = jnp.mean(t, axis=0, keepdims=True)
    v = jnp.mean(t * t, axis=0, keepdims=True) - m * m
    return jnp.maximum((t - m) * lax.rsqrt(v + BN_EPS) * g + b, 0.0)


def _layer_body(h_ref, agg_ref, eps_ref, w1_ref, b1_ref, g1_ref, be1_ref,
                w2_ref, b2_ref, gbn_ref, bbn_ref, batch_ref,
                hout_ref, pool_ref):
    z = (1.0 + eps_ref[0, 0]) * h_ref[...] + agg_ref[0, :N, :] + agg_ref[1, :N, :]
    t = jnp.dot(z, w1_ref[...], preferred_element_type=jnp.float32) + b1_ref[...]
    t = _bn_relu(t, g1_ref[...], be1_ref[...])
    u = jnp.dot(t, w2_ref[...], preferred_element_type=jnp.float32) + b2_ref[...]
    hn = _bn_relu(u, gbn_ref[...], bbn_ref[...])
    hout_ref[...] = hn
    pool_ref[...] = _pool(batch_ref[...], hn)


def _final_body(pools_ref, w_ref, b_ref, out_ref):
    p = pools_ref[...]
    pc = jnp.concatenate([p[i] for i in range(L + 1)], axis=1)
    out_ref[...] = jnp.dot(pc, w_ref[...],
                           preferred_element_type=jnp.float32) + b_ref[...]


@functools.cache
def _inproj_call():
    return pl.pallas_call(
        _inproj_body,
        out_shape=(jax.ShapeDtypeStruct((N, H), jnp.float32),
                   jax.ShapeDtypeStruct((G, H), jnp.float32)),
    )


@functools.cache
def _layer_call():
    return pl.pallas_call(
        _layer_body,
        out_shape=(jax.ShapeDtypeStruct((N, H), jnp.float32),
                   jax.ShapeDtypeStruct((G, H), jnp.float32)),
    )


@functools.cache
def _final_call():
    return pl.pallas_call(
        _final_body,
        out_shape=jax.ShapeDtypeStruct((G, H), jnp.float32),
    )


def kernel(x, edge_index, batch, params):
    pad = E_PAD - E
    src = jnp.concatenate(
        [edge_index[0], jnp.zeros((pad,), jnp.int32)]).reshape(NW, NCHUNK, CHUNK)
    dst = jnp.concatenate(
        [edge_index[1], jnp.full((pad,), N, jnp.int32)]).reshape(NW, NCHUNK, CHUNK)
    batch2 = batch.reshape(N, 1)

    h, pool0 = _inproj_call()(
        x, params['in_proj']['w'], params['in_proj']['b'].reshape(1, H), batch2)
    pools = [pool0]
    for lp in params['layers']:
        agg2 = _segsum_call()(h, src, dst)
        h, p = _layer_call()(
            h, agg2, lp['eps'].reshape(1, 1),
            lp['w1'], lp['b1'].reshape(1, 2 * H),
            lp['g1'].reshape(1, 2 * H), lp['be1'].reshape(1, 2 * H),
            lp['w2'], lp['b2'].reshape(1, H),
            lp['g_bn'].reshape(1, H), lp['b_bn'].reshape(1, H), batch2)
        pools.append(p)

    return _final_call()(jnp.stack(pools),
                         params['out_proj']['w'],
                         params['out_proj']['b'].reshape(1, H))


# speedup vs baseline: 3.7195x; 1.4662x over previous
import functools

import jax
import jax.numpy as jnp
from jax import lax
from jax.experimental import pallas as pl
from jax.experimental.pallas import tpu as pltpu
from jax.experimental.pallas import tpu_sc as plsc

N = 10000
E = 320000
H = 128
G = 64
L = 5
BN_EPS = 1e-5

NC = 2
NS = 16
NW = NC * NS
CHUNK = 80
NCHUNK = 125
EPW = NCHUNK * CHUNK
E_PAD = NW * EPW
N_PAD = 10016
ROWS_PER_TILE = N_PAD // NS


def _segsum_body(h_hbm, src_hbm, dst_hbm, out_hbm,
                 src_v, dst_v, rows_a, rows_b, rows_c, agg_sh,
                 sem_a, sem_b, sem_c):
    cid = lax.axis_index("c")
    sid = lax.axis_index("s")
    wid = cid * NS + sid

    pltpu.async_copy(src_hbm.at[wid], src_v, sem_a)
    pltpu.async_copy(dst_hbm.at[wid], dst_v, sem_b)

    def _zero_row(i, carry):
        for c in range(H // 16):
            rows_a[i, pl.ds(c * 16, 16)] = jnp.zeros((16,), jnp.float32)
        return carry
    lax.fori_loop(0, CHUNK, _zero_row, 0)
    row0 = sid * ROWS_PER_TILE
    for k in range(ROWS_PER_TILE // CHUNK):
        pltpu.sync_copy(rows_a, agg_sh.at[pl.ds(row0 + k * CHUNK, CHUNK)])
    rem = ROWS_PER_TILE - (ROWS_PER_TILE // CHUNK) * CHUNK
    if rem:
        pltpu.sync_copy(
            rows_a.at[pl.ds(0, rem)],
            agg_sh.at[pl.ds(row0 + (ROWS_PER_TILE // CHUNK) * CHUNK, rem)])
    pltpu.make_async_copy(src_hbm.at[wid], src_v, sem_a).wait()
    pltpu.make_async_copy(dst_hbm.at[wid], dst_v, sem_b).wait()
    plsc.subcore_barrier()

    pltpu.async_copy(h_hbm.at[src_v.at[0]], rows_a, sem_a)
    pltpu.async_copy(h_hbm.at[src_v.at[1]], rows_b, sem_b)

    def _triple(i, carry):
        j0 = 3 * i
        pltpu.make_async_copy(h_hbm.at[src_v.at[j0]], rows_a, sem_a).wait()
        pltpu.async_copy(h_hbm.at[src_v.at[j0 + 2]], rows_c, sem_c)
        pltpu.sync_copy(rows_a, agg_sh.at[dst_v.at[j0]], add=True)
        pltpu.make_async_copy(h_hbm.at[src_v.at[j0 + 1]], rows_b, sem_b).wait()
        pltpu.async_copy(h_hbm.at[src_v.at[j0 + 3]], rows_a, sem_a)
        pltpu.sync_copy(rows_b, agg_sh.at[dst_v.at[j0 + 1]], add=True)
        pltpu.make_async_copy(h_hbm.at[src_v.at[j0 + 2]], rows_c, sem_c).wait()
        pltpu.async_copy(h_hbm.at[src_v.at[j0 + 4]], rows_b, sem_b)
        pltpu.sync_copy(rows_c, agg_sh.at[dst_v.at[j0 + 2]], add=True)
        return carry
    lax.fori_loop(0, (NCHUNK - 2) // 3, _triple, 0)
    pltpu.make_async_copy(h_hbm.at[src_v.at[NCHUNK - 2]], rows_a, sem_a).wait()
    pltpu.sync_copy(rows_a, agg_sh.at[dst_v.at[NCHUNK - 2]], add=True)
    pltpu.make_async_copy(h_hbm.at[src_v.at[NCHUNK - 1]], rows_b, sem_b).wait()
    pltpu.sync_copy(rows_b, agg_sh.at[dst_v.at[NCHUNK - 1]], add=True)
    plsc.subcore_barrier()

    pltpu.sync_copy(agg_sh.at[pl.ds(row0, ROWS_PER_TILE)],
                    out_hbm.at[cid, pl.ds(row0, ROWS_PER_TILE)])


@functools.cache
def _segsum_call():
    return pl.kernel(
        _segsum_body,
        compiler_params=pltpu.CompilerParams(use_tc_tiling_on_sc=False),
        out_type=jax.ShapeDtypeStruct((NC, N_PAD, H), jnp.float32),
        mesh=plsc.VectorSubcoreMesh(core_axis_name="c", subcore_axis_name="s",
                                    num_cores=NC, num_subcores=NS),
        scratch_types=[
            pltpu.VMEM((NCHUNK, CHUNK), jnp.int32),
            pltpu.VMEM((NCHUNK, CHUNK), jnp.int32),
            pltpu.VMEM((CHUNK, H), jnp.float32),
            pltpu.VMEM((CHUNK, H), jnp.float32),
            pltpu.VMEM((CHUNK, H), jnp.float32),
            pltpu.VMEM_SHARED((N_PAD, H), jnp.float32),
            pltpu.SemaphoreType.DMA,
            pltpu.SemaphoreType.DMA,
            pltpu.SemaphoreType.DMA,
        ],
    )


def _pool(batch2, h):
    onehot = (batch2 == lax.broadcasted_iota(jnp.int32, (N, G), 1))
    return lax.dot_general(onehot.astype(jnp.float32), h,
                           (((0,), (0,)), ((), ())),
                           preferred_element_type=jnp.float32)


def _inproj_body(x_ref, w_ref, b_ref, batch_ref, h_ref, pool_ref):
    h = jnp.dot(x_ref[...], w_ref[...],
                preferred_element_type=jnp.float32) + b_ref[...]
    h_ref[...] = h
    pool_ref[...] = _pool(batch_ref[...], h)


def _bn_relu(t, g, b):
    m = jnp.mean(t, axis=0, keepdims=True)
    v = jnp.mean(t * t, axis=0, keepdims=True) - m * m
    return jnp.maximum((t - m) * lax.rsqrt(v + BN_EPS) * g + b, 0.0)


def _layer_body(h_ref, agg_ref, eps_ref, w1_ref, b1_ref, g1_ref, be1_ref,
                w2_ref, b2_ref, gbn_ref, bbn_ref, batch_ref,
                hout_ref, pool_ref):
    z = (1.0 + eps_ref[0, 0]) * h_ref[...] + agg_ref[0, :N, :] + agg_ref[1, :N, :]
    t = jnp.dot(z, w1_ref[...], preferred_element_type=jnp.float32) + b1_ref[...]
    t = _bn_relu(t, g1_ref[...], be1_ref[...])
    u = jnp.dot(t, w2_ref[...], preferred_element_type=jnp.float32) + b2_ref[...]
    hn = _bn_relu(u, gbn_ref[...], bbn_ref[...])
    hout_ref[...] = hn
    pool_ref[...] = _pool(batch_ref[...], hn)


def _final_body(pools_ref, w_ref, b_ref, out_ref):
    p = pools_ref[...]
    pc = jnp.concatenate([p[i] for i in range(L + 1)], axis=1)
    out_ref[...] = jnp.dot(pc, w_ref[...],
                           preferred_element_type=jnp.float32) + b_ref[...]


@functools.cache
def _inproj_call():
    return pl.pallas_call(
        _inproj_body,
        out_shape=(jax.ShapeDtypeStruct((N, H), jnp.float32),
                   jax.ShapeDtypeStruct((G, H), jnp.float32)),
    )


@functools.cache
def _layer_call():
    return pl.pallas_call(
        _layer_body,
        out_shape=(jax.ShapeDtypeStruct((N, H), jnp.float32),
                   jax.ShapeDtypeStruct((G, H), jnp.float32)),
    )


@functools.cache
def _final_call():
    return pl.pallas_call(
        _final_body,
        out_shape=jax.ShapeDtypeStruct((G, H), jnp.float32),
    )


def kernel(x, edge_index, batch, params):
    pad = E_PAD - E
    src = jnp.concatenate(
        [edge_index[0], jnp.zeros((pad,), jnp.int32)]).reshape(NW, NCHUNK, CHUNK)
    dst = jnp.concatenate(
        [edge_index[1], jnp.full((pad,), N, jnp.int32)]).reshape(NW, NCHUNK, CHUNK)
    batch2 = batch.reshape(N, 1)

    h, pool0 = _inproj_call()(
        x, params['in_proj']['w'], params['in_proj']['b'].reshape(1, H), batch2)
    pools = [pool0]
    for lp in params['layers']:
        agg2 = _segsum_call()(h, src, dst)
        h, p = _layer_call()(
            h, agg2, lp['eps'].reshape(1, 1),
            lp['w1'], lp['b1'].reshape(1, 2 * H),
            lp['g1'].reshape(1, 2 * H), lp['be1'].reshape(1, 2 * H),
            lp['w2'], lp['b2'].reshape(1, H),
            lp['g_bn'].reshape(1, H), lp['b_bn'].reshape(1, H), batch2)
        pools.append(p)

    return _final_call()(jnp.stack(pools),
                         params['out_proj']['w'],
                         params['out_proj']['b'].reshape(1, H))


# speedup vs baseline: 3.7415x; 1.0059x over previous
import functools

import jax
import jax.numpy as jnp
from jax import lax
from jax.experimental import pallas as pl
from jax.experimental.pallas import tpu as pltpu
from jax.experimental.pallas import tpu_sc as plsc

N = 10000
E = 320000
H = 128
G = 64
L = 5
BN_EPS = 1e-5

NC = 2
NS = 16
NW = NC * NS
CHUNK = 80
NCHUNK = 125
EPW = NCHUNK * CHUNK
E_PAD = NW * EPW
N_PAD = 10016
ROWS_PER_TILE = N_PAD // NS


def _segsum_body(h_hbm, src_hbm, dst_hbm, out_hbm,
                 src_v, dst_v, rows_a, rows_b, rows_c, agg_sh,
                 sem_a, sem_b, sem_c):
    cid = lax.axis_index("c")
    sid = lax.axis_index("s")
    wid = cid * NS + sid

    pltpu.async_copy(src_hbm.at[wid], src_v, sem_a)
    pltpu.async_copy(dst_hbm.at[wid], dst_v, sem_b)

    def _zero_row(i, carry):
        for c in range(H // 16):
            rows_a[i, pl.ds(c * 16, 16)] = jnp.zeros((16,), jnp.float32)
        return carry
    lax.fori_loop(0, CHUNK, _zero_row, 0)
    row0 = sid * ROWS_PER_TILE
    nfull = ROWS_PER_TILE // CHUNK
    for k in range(nfull):
        pltpu.async_copy(rows_a, agg_sh.at[pl.ds(row0 + k * CHUNK, CHUNK)],
                         sem_c)
    rem = ROWS_PER_TILE - nfull * CHUNK
    if rem:
        pltpu.async_copy(rows_a.at[pl.ds(0, rem)],
                         agg_sh.at[pl.ds(row0 + nfull * CHUNK, rem)], sem_c)
    for k in range(nfull):
        pltpu.make_async_copy(rows_a, agg_sh.at[pl.ds(row0 + k * CHUNK, CHUNK)],
                              sem_c).wait()
    if rem:
        pltpu.make_async_copy(rows_a.at[pl.ds(0, rem)],
                              agg_sh.at[pl.ds(row0 + nfull * CHUNK, rem)],
                              sem_c).wait()
    pltpu.make_async_copy(src_hbm.at[wid], src_v, sem_a).wait()
    pltpu.make_async_copy(dst_hbm.at[wid], dst_v, sem_b).wait()
    plsc.subcore_barrier()

    pltpu.async_copy(h_hbm.at[src_v.at[0]], rows_a, sem_a)
    pltpu.async_copy(h_hbm.at[src_v.at[1]], rows_b, sem_b)

    def _triple(i, carry):
        j0 = 3 * i
        pltpu.make_async_copy(h_hbm.at[src_v.at[j0]], rows_a, sem_a).wait()
        pltpu.async_copy(h_hbm.at[src_v.at[j0 + 2]], rows_c, sem_c)
        pltpu.sync_copy(rows_a, agg_sh.at[dst_v.at[j0]], add=True)
        pltpu.make_async_copy(h_hbm.at[src_v.at[j0 + 1]], rows_b, sem_b).wait()
        pltpu.async_copy(h_hbm.at[src_v.at[j0 + 3]], rows_a, sem_a)
        pltpu.sync_copy(rows_b, agg_sh.at[dst_v.at[j0 + 1]], add=True)
        pltpu.make_async_copy(h_hbm.at[src_v.at[j0 + 2]], rows_c, sem_c).wait()
        pltpu.async_copy(h_hbm.at[src_v.at[j0 + 4]], rows_b, sem_b)
        pltpu.sync_copy(rows_c, agg_sh.at[dst_v.at[j0 + 2]], add=True)
        return carry
    lax.fori_loop(0, (NCHUNK - 2) // 3, _triple, 0)
    pltpu.make_async_copy(h_hbm.at[src_v.at[NCHUNK - 2]], rows_a, sem_a).wait()
    pltpu.sync_copy(rows_a, agg_sh.at[dst_v.at[NCHUNK - 2]], add=True)
    pltpu.make_async_copy(h_hbm.at[src_v.at[NCHUNK - 1]], rows_b, sem_b).wait()
    pltpu.sync_copy(rows_b, agg_sh.at[dst_v.at[NCHUNK - 1]], add=True)
    plsc.subcore_barrier()

    pltpu.sync_copy(agg_sh.at[pl.ds(row0, ROWS_PER_TILE)],
                    out_hbm.at[cid, pl.ds(row0, ROWS_PER_TILE)])


@functools.cache
def _segsum_call():
    return pl.kernel(
        _segsum_body,
        compiler_params=pltpu.CompilerParams(use_tc_tiling_on_sc=False),
        out_type=jax.ShapeDtypeStruct((NC, N_PAD, H), jnp.float32),
        mesh=plsc.VectorSubcoreMesh(core_axis_name="c", subcore_axis_name="s",
                                    num_cores=NC, num_subcores=NS),
        scratch_types=[
            pltpu.VMEM((NCHUNK, CHUNK), jnp.int32),
            pltpu.VMEM((NCHUNK, CHUNK), jnp.int32),
            pltpu.VMEM((CHUNK, H), jnp.float32),
            pltpu.VMEM((CHUNK, H), jnp.float32),
            pltpu.VMEM((CHUNK, H), jnp.float32),
            pltpu.VMEM_SHARED((N_PAD, H), jnp.float32),
            pltpu.SemaphoreType.DMA,
            pltpu.SemaphoreType.DMA,
            pltpu.SemaphoreType.DMA,
        ],
    )


def _pool(batch2, h):
    onehot = (batch2 == lax.broadcasted_iota(jnp.int32, (N, G), 1))
    return lax.dot_general(onehot.astype(jnp.float32), h,
                           (((0,), (0,)), ((), ())),
                           preferred_element_type=jnp.float32)


def _inproj_body(x_ref, w_ref, b_ref, batch_ref, h_ref, pool_ref):
    h = jnp.dot(x_ref[...], w_ref[...],
                preferred_element_type=jnp.float32) + b_ref[...]
    h_ref[...] = h
    pool_ref[...] = _pool(batch_ref[...], h)


def _bn_relu(t, g, b):
    m = jnp.mean(t, axis=0, keepdims=True)
    v = jnp.mean(t * t, axis=0, keepdims=True) - m * m
    return jnp.maximum((t - m) * lax.rsqrt(v + BN_EPS) * g + b, 0.0)


def _layer_body(h_ref, agg_ref, eps_ref, w1_ref, b1_ref, g1_ref, be1_ref,
                w2_ref, b2_ref, gbn_ref, bbn_ref, batch_ref,
                hout_ref, pool_ref):
    z = (1.0 + eps_ref[0, 0]) * h_ref[...] + agg_ref[0, :N, :] + agg_ref[1, :N, :]
    t = jnp.dot(z, w1_ref[...], preferred_element_type=jnp.float32) + b1_ref[...]
    t = _bn_relu(t, g1_ref[...], be1_ref[...])
    u = jnp.dot(t, w2_ref[...], preferred_element_type=jnp.float32) + b2_ref[...]
    hn = _bn_relu(u, gbn_ref[...], bbn_ref[...])
    hout_ref[...] = hn
    pool_ref[...] = _pool(batch_ref[...], hn)


def _final_body(pools_ref, w_ref, b_ref, out_ref):
    p = pools_ref[...]
    pc = jnp.concatenate([p[i] for i in range(L + 1)], axis=1)
    out_ref[...] = jnp.dot(pc, w_ref[...],
                           preferred_element_type=jnp.float32) + b_ref[...]


@functools.cache
def _inproj_call():
    return pl.pallas_call(
        _inproj_body,
        out_shape=(jax.ShapeDtypeStruct((N, H), jnp.float32),
                   jax.ShapeDtypeStruct((G, H), jnp.float32)),
    )


@functools.cache
def _layer_call():
    return pl.pallas_call(
        _layer_body,
        out_shape=(jax.ShapeDtypeStruct((N, H), jnp.float32),
                   jax.ShapeDtypeStruct((G, H), jnp.float32)),
    )


@functools.cache
def _final_call():
    return pl.pallas_call(
        _final_body,
        out_shape=jax.ShapeDtypeStruct((G, H), jnp.float32),
    )


def kernel(x, edge_index, batch, params):
    pad = E_PAD - E
    src = jnp.concatenate(
        [edge_index[0], jnp.zeros((pad,), jnp.int32)]).reshape(NW, NCHUNK, CHUNK)
    dst = jnp.concatenate(
        [edge_index[1], jnp.full((pad,), N, jnp.int32)]).reshape(NW, NCHUNK, CHUNK)
    batch2 = batch.reshape(N, 1)

    h, pool0 = _inproj_call()(
        x, params['in_proj']['w'], params['in_proj']['b'].reshape(1, H), batch2)
    pools = [pool0]
    for lp in params['layers']:
        agg2 = _segsum_call()(h, src, dst)
        h, p = _layer_call()(
            h, agg2, lp['eps'].reshape(1, 1),
            lp['w1'], lp['b1'].reshape(1, 2 * H),
            lp['g1'].reshape(1, 2 * H), lp['be1'].reshape(1, 2 * H),
            lp['w2'], lp['b2'].reshape(1, H),
            lp['g_bn'].reshape(1, H), lp['b_bn'].reshape(1, H), batch2)
        pools.append(p)

    return _final_call()(jnp.stack(pools),
                         params['out_proj']['w'],
                         params['out_proj']['b'].reshape(1, H))


# speedup vs baseline: 3.7460x; 1.0012x over previous
import functools

import jax
import jax.numpy as jnp
from jax import lax
from jax.experimental import pallas as pl
from jax.experimental.pallas import tpu as pltpu
from jax.experimental.pallas import tpu_sc as plsc

N = 10000
E = 320000
H = 128
G = 64
L = 5
BN_EPS = 1e-5

NC = 2
NS = 16
NW = NC * NS
CHUNK = 80
NCHUNK = 125
EPW = NCHUNK * CHUNK
E_PAD = NW * EPW
N_PAD = 10016
ROWS_PER_TILE = N_PAD // NS


def _segsum_body(h_hbm, src_hbm, dst_hbm, out_hbm,
                 src_v, dst_v, rows_a, rows_b, rows_c, agg_sh,
                 sem_a, sem_b, sem_c):
    cid = lax.axis_index("c")
    sid = lax.axis_index("s")
    wid = cid * NS + sid

    pltpu.async_copy(src_hbm.at[wid], src_v, sem_a)
    pltpu.async_copy(dst_hbm.at[wid], dst_v, sem_b)

    def _zero_row(i, carry):
        for c in range(H // 16):
            rows_a[i, pl.ds(c * 16, 16)] = jnp.zeros((16,), jnp.float32)
        return carry
    lax.fori_loop(0, CHUNK, _zero_row, 0)
    row0 = sid * ROWS_PER_TILE
    nfull = ROWS_PER_TILE // CHUNK
    for k in range(nfull):
        pltpu.async_copy(rows_a, agg_sh.at[pl.ds(row0 + k * CHUNK, CHUNK)],
                         sem_c)
    rem = ROWS_PER_TILE - nfull * CHUNK
    if rem:
        pltpu.async_copy(rows_a.at[pl.ds(0, rem)],
                         agg_sh.at[pl.ds(row0 + nfull * CHUNK, rem)], sem_c)
    for k in range(nfull):
        pltpu.make_async_copy(rows_a, agg_sh.at[pl.ds(row0 + k * CHUNK, CHUNK)],
                              sem_c).wait()
    if rem:
        pltpu.make_async_copy(rows_a.at[pl.ds(0, rem)],
                              agg_sh.at[pl.ds(row0 + nfull * CHUNK, rem)],
                              sem_c).wait()
    pltpu.make_async_copy(src_hbm.at[wid], src_v, sem_a).wait()
    pltpu.make_async_copy(dst_hbm.at[wid], dst_v, sem_b).wait()
    plsc.subcore_barrier()

    pltpu.async_copy(h_hbm.at[src_v.at[0]], rows_a, sem_a)
    pltpu.async_copy(h_hbm.at[src_v.at[1]], rows_b, sem_b)

    def _triple(i, carry):
        j0 = 3 * i
        pltpu.make_async_copy(h_hbm.at[src_v.at[j0]], rows_a, sem_a).wait()
        pltpu.async_copy(h_hbm.at[src_v.at[j0 + 2]], rows_c, sem_c)
        pltpu.sync_copy(rows_a, agg_sh.at[dst_v.at[j0]], add=True)
        pltpu.make_async_copy(h_hbm.at[src_v.at[j0 + 1]], rows_b, sem_b).wait()
        pltpu.async_copy(h_hbm.at[src_v.at[j0 + 3]], rows_a, sem_a)
        pltpu.sync_copy(rows_b, agg_sh.at[dst_v.at[j0 + 1]], add=True)
        pltpu.make_async_copy(h_hbm.at[src_v.at[j0 + 2]], rows_c, sem_c).wait()
        pltpu.async_copy(h_hbm.at[src_v.at[j0 + 4]], rows_b, sem_b)
        pltpu.sync_copy(rows_c, agg_sh.at[dst_v.at[j0 + 2]], add=True)
        return carry
    lax.fori_loop(0, (NCHUNK - 2) // 3, _triple, 0)
    pltpu.make_async_copy(h_hbm.at[src_v.at[NCHUNK - 2]], rows_a, sem_a).wait()
    pltpu.sync_copy(rows_a, agg_sh.at[dst_v.at[NCHUNK - 2]], add=True)
    pltpu.make_async_copy(h_hbm.at[src_v.at[NCHUNK - 1]], rows_b, sem_b).wait()
    pltpu.sync_copy(rows_b, agg_sh.at[dst_v.at[NCHUNK - 1]], add=True)
    plsc.subcore_barrier()

    pltpu.sync_copy(agg_sh.at[pl.ds(row0, ROWS_PER_TILE)],
                    out_hbm.at[cid, pl.ds(row0, ROWS_PER_TILE)])


@functools.cache
def _segsum_call():
    return pl.kernel(
        _segsum_body,
        compiler_params=pltpu.CompilerParams(use_tc_tiling_on_sc=False),
        out_type=jax.ShapeDtypeStruct((NC, N_PAD, H), jnp.float32),
        mesh=plsc.VectorSubcoreMesh(core_axis_name="c", subcore_axis_name="s",
                                    num_cores=NC, num_subcores=NS),
        scratch_types=[
            pltpu.VMEM((NCHUNK, CHUNK), jnp.int32),
            pltpu.VMEM((NCHUNK, CHUNK), jnp.int32),
            pltpu.VMEM((CHUNK, H), jnp.float32),
            pltpu.VMEM((CHUNK, H), jnp.float32),
            pltpu.VMEM((CHUNK, H), jnp.float32),
            pltpu.VMEM_SHARED((N_PAD, H), jnp.float32),
            pltpu.SemaphoreType.DMA,
            pltpu.SemaphoreType.DMA,
            pltpu.SemaphoreType.DMA,
        ],
    )


def _pool(batch2, h):
    onehot = (batch2 == lax.broadcasted_iota(jnp.int32, (N, G), 1))
    return lax.dot_general(onehot.astype(jnp.float32), h,
                           (((0,), (0,)), ((), ())),
                           preferred_element_type=jnp.float32)


def _inproj_body(x_ref, w_ref, b_ref, batch_ref, h_ref, pool_ref):
    h = jnp.dot(x_ref[...], w_ref[...],
                preferred_element_type=jnp.float32) + b_ref[...]
    h_ref[...] = h
    pool_ref[...] = _pool(batch_ref[...], h)


def _bn_relu(t, g, b):
    m = jnp.mean(t, axis=0, keepdims=True)
    v = jnp.mean(t * t, axis=0, keepdims=True) - m * m
    return jnp.maximum((t - m) * lax.rsqrt(v + BN_EPS) * g + b, 0.0)


def _layer_body(h_ref, agg_ref, eps_ref, w1_ref, b1_ref, g1_ref, be1_ref,
                w2_ref, b2_ref, gbn_ref, bbn_ref, batch_ref,
                hout_ref, pool_ref):
    z = (1.0 + eps_ref[0, 0]) * h_ref[...] + agg_ref[0, :N, :] + agg_ref[1, :N, :]
    t = jnp.dot(z, w1_ref[...], preferred_element_type=jnp.float32) + b1_ref[...]
    t = _bn_relu(t, g1_ref[...], be1_ref[...])
    u = jnp.dot(t, w2_ref[...], preferred_element_type=jnp.float32) + b2_ref[...]
    hn = _bn_relu(u, gbn_ref[...], bbn_ref[...])
    hout_ref[...] = hn
    pool_ref[...] = _pool(batch_ref[...], hn)


def _last_body(h_ref, agg_ref, eps_ref, w1_ref, b1_ref, g1_ref, be1_ref,
               w2_ref, b2_ref, gbn_ref, bbn_ref, batch_ref,
               pools_ref, wout_ref, bout_ref, out_ref):
    z = (1.0 + eps_ref[0, 0]) * h_ref[...] + agg_ref[0, :N, :] + agg_ref[1, :N, :]
    t = jnp.dot(z, w1_ref[...], preferred_element_type=jnp.float32) + b1_ref[...]
    t = _bn_relu(t, g1_ref[...], be1_ref[...])
    u = jnp.dot(t, w2_ref[...], preferred_element_type=jnp.float32) + b2_ref[...]
    hn = _bn_relu(u, gbn_ref[...], bbn_ref[...])
    p5 = _pool(batch_ref[...], hn)
    p = pools_ref[...]
    pc = jnp.concatenate([p[i] for i in range(L)] + [p5], axis=1)
    out_ref[...] = jnp.dot(pc, wout_ref[...],
                           preferred_element_type=jnp.float32) + bout_ref[...]


@functools.cache
def _inproj_call():
    return pl.pallas_call(
        _inproj_body,
        out_shape=(jax.ShapeDtypeStruct((N, H), jnp.float32),
                   jax.ShapeDtypeStruct((G, H), jnp.float32)),
    )


@functools.cache
def _layer_call():
    return pl.pallas_call(
        _layer_body,
        out_shape=(jax.ShapeDtypeStruct((N, H), jnp.float32),
                   jax.ShapeDtypeStruct((G, H), jnp.float32)),
    )


@functools.cache
def _last_call():
    return pl.pallas_call(
        _last_body,
        out_shape=jax.ShapeDtypeStruct((G, H), jnp.float32),
    )


def kernel(x, edge_index, batch, params):
    pad = E_PAD - E
    src = jnp.concatenate(
        [edge_index[0], jnp.zeros((pad,), jnp.int32)]).reshape(NW, NCHUNK, CHUNK)
    dst = jnp.concatenate(
        [edge_index[1], jnp.full((pad,), N, jnp.int32)]).reshape(NW, NCHUNK, CHUNK)
    batch2 = batch.reshape(N, 1)

    h, pool0 = _inproj_call()(
        x, params['in_proj']['w'], params['in_proj']['b'].reshape(1, H), batch2)
    pools = [pool0]
    for lp in params['layers'][:-1]:
        agg2 = _segsum_call()(h, src, dst)
        h, p = _layer_call()(
            h, agg2, lp['eps'].reshape(1, 1),
            lp['w1'], lp['b1'].reshape(1, 2 * H),
            lp['g1'].reshape(1, 2 * H), lp['be1'].reshape(1, 2 * H),
            lp['w2'], lp['b2'].reshape(1, H),
            lp['g_bn'].reshape(1, H), lp['b_bn'].reshape(1, H), batch2)
        pools.append(p)

    lp = params['layers'][-1]
    agg2 = _segsum_call()(h, src, dst)
    return _last_call()(
        h, agg2, lp['eps'].reshape(1, 1),
        lp['w1'], lp['b1'].reshape(1, 2 * H),
        lp['g1'].reshape(1, 2 * H), lp['be1'].reshape(1, 2 * H),
        lp['w2'], lp['b2'].reshape(1, H),
        lp['g_bn'].reshape(1, H), lp['b_bn'].reshape(1, H), batch2,
        jnp.stack(pools), params['out_proj']['w'],
        params['out_proj']['b'].reshape(1, H))
